# baseline (speedup 1.0000x reference)
import functools

import jax
import jax.numpy as jnp
from jax import lax
from jax.experimental import pallas as pl
from jax.experimental.pallas import tpu as pltpu
from jax.experimental.pallas import tpu_sc as plsc

N = 10000
D = 128
NC = 2
NS = 16
NW = NC * NS
N_PAD = 10240
ROWS_PER_SUB = N_PAD // NS
CHUNK = 128
IBLK = 8
R_BLK = 512
FAST_CORE = 1
FAST_FRAC_NUM, FAST_FRAC_DEN = 1, 1


def _make_deg_kernel(total_rows):
    mesh = plsc.VectorSubcoreMesh(core_axis_name="c", subcore_axis_name="s")
    rows_per_sub = total_rows // NS

    @functools.partial(
        pl.kernel,
        mesh=mesh,
        out_type=jax.ShapeDtypeStruct((NC, N_PAD, D), jnp.float32),
        scratch_types=[
            pltpu.VMEM_SHARED((N_PAD, D), jnp.float32),
            pltpu.VMEM((CHUNK, D), jnp.float32),
            pltpu.VMEM((IBLK, CHUNK), jnp.int32),
            pltpu.SemaphoreType.DMA,
        ],
    )
    def deg_kernel(src_hbm, dst_hbm, cnt_hbm, cnt_sh, ones_v, idx_blk, ssem):
        c = lax.axis_index("c")
        s = lax.axis_index("s")

        def fill(val):
            def body(i, _):
                for k in range(D // 16):
                    ones_v[i, pl.ds(k * 16, 16)] = jnp.full((16,), val,
                                                            jnp.float32)
                return 0
            lax.fori_loop(0, CHUNK, body, 0)

        fill(0.0)
        base = s * ROWS_PER_SUB
        for k in range(ROWS_PER_SUB // CHUNK):
            pltpu.sync_copy(ones_v, cnt_sh.at[pl.ds(base + k * CHUNK, CHUNK)])
        fill(1.0)
        plsc.subcore_barrier()

        def count(edge_hbm):
            def body(b, _):
                rb = s * rows_per_sub + b * IBLK
                pltpu.sync_copy(edge_hbm.at[pl.ds(rb, IBLK)], idx_blk)
                cps = [pltpu.async_copy(ones_v, cnt_sh.at[idx_blk.at[j]],
                                        ssem, add=True)
                       for j in range(IBLK)]
                for cp in cps:
                    cp.wait()
                return 0
            lax.fori_loop(0, rows_per_sub // IBLK, body, 0)

        @pl.when(c == 0)
        def _():
            count(src_hbm)

        @pl.when(c == 1)
        def _():
            count(dst_hbm)

        plsc.subcore_barrier()
        for k in range(ROWS_PER_SUB // CHUNK):
            pltpu.sync_copy(cnt_sh.at[pl.ds(base + k * CHUNK, CHUNK)], ones_v)
            pltpu.sync_copy(ones_v,
                            cnt_hbm.at[c].at[pl.ds(base + k * CHUNK, CHUNK)])

    return deg_kernel


def _make_agg_kernel(total_rows):
    mesh = plsc.VectorSubcoreMesh(core_axis_name="c", subcore_axis_name="s")
    grain = NS * IBLK
    fast_total = (total_rows * FAST_FRAC_NUM // FAST_FRAC_DEN) // grain * grain
    f_t = fast_total // NS
    s_t = (total_rows - fast_total) // NS

    @functools.partial(
        pl.kernel,
        mesh=mesh,
        out_type=jax.ShapeDtypeStruct((NC, N_PAD, D), jnp.float32),
        scratch_types=[
            pltpu.VMEM_SHARED((N_PAD, D), jnp.float32),
            pltpu.VMEM((CHUNK, D), jnp.float32),
            pltpu.VMEM((CHUNK, D), jnp.float32),
            pltpu.VMEM((IBLK, CHUNK), jnp.int32),
            pltpu.VMEM((IBLK, CHUNK), jnp.int32),
            pltpu.SemaphoreType.DMA,
            pltpu.SemaphoreType.DMA,
            pltpu.SemaphoreType.DMA,
            pltpu.SemaphoreType.DMA,
        ],
    )
    def agg_kernel(table_hbm, src_hbm, dst_hbm, out_hbm,
                   acc_sh, rows0, rows1, sidx_blk, didx_blk,
                   gsem0, gsem1, ssem0, ssem1):
        c = lax.axis_index("c")
        s = lax.axis_index("s")
        my_rows = jnp.where(c == FAST_CORE, f_t, s_t)
        my_base = jnp.where(c == FAST_CORE, s * f_t, fast_total + s * s_t)

        def zrow(i, _):
            for k in range(D // 16):
                rows0[i, pl.ds(k * 16, 16)] = jnp.zeros((16,), jnp.float32)
            return 0
        lax.fori_loop(0, CHUNK, zrow, 0)

        base = s * ROWS_PER_SUB
        for k in range(ROWS_PER_SUB // CHUNK):
            pltpu.sync_copy(rows0, acc_sh.at[pl.ds(base + k * CHUNK, CHUNK)])
        plsc.subcore_barrier()

        def body(b, _):
            @pl.when(b > 0)
            def _():
                pltpu.make_async_copy(table_hbm.at[sidx_blk.at[0]], rows0,
                                      ssem0).wait()
                pltpu.make_async_copy(table_hbm.at[sidx_blk.at[0]], rows1,
                                      ssem1).wait()
            rb = my_base + b * IBLK
            pltpu.sync_copy(src_hbm.at[pl.ds(rb, IBLK)], sidx_blk)
            pltpu.sync_copy(dst_hbm.at[pl.ds(rb, IBLK)], didx_blk)
            for jj in range(IBLK // 2):
                j0, j1 = 2 * jj, 2 * jj + 1
                if jj > 0:
                    pltpu.make_async_copy(table_hbm.at[sidx_blk.at[j0]],
                                          rows0, ssem0).wait()
                    pltpu.make_async_copy(table_hbm.at[sidx_blk.at[j1]],
                                          rows1, ssem1).wait()
                cp0 = pltpu.async_copy(table_hbm.at[sidx_blk.at[j0]], rows0,
                                       gsem0)
                cp1 = pltpu.async_copy(table_hbm.at[sidx_blk.at[j1]], rows1,
                                       gsem1)
                cp0.wait()
                pltpu.async_copy(rows0, acc_sh.at[didx_blk.at[j0]], ssem0,
                                 add=True)
                cp1.wait()
                pltpu.async_copy(rows1, acc_sh.at[didx_blk.at[j1]], ssem1,
                                 add=True)
            return 0
        lax.fori_loop(0, my_rows // IBLK, body, 0)

        @pl.when(my_rows > 0)
        def _():
            pltpu.make_async_copy(table_hbm.at[sidx_blk.at[0]], rows0,
                                  ssem0).wait()
            pltpu.make_async_copy(table_hbm.at[sidx_blk.at[0]], rows1,
                                  ssem1).wait()
        plsc.subcore_barrier()

        for k in range(ROWS_PER_SUB // CHUNK):
            pltpu.sync_copy(acc_sh.at[pl.ds(base + k * CHUNK, CHUNK)], rows0)
            pltpu.sync_copy(rows0,
                            out_hbm.at[c].at[pl.ds(base + k * CHUNK, CHUNK)])

    return agg_kernel


def _tc_prep(cnt, feat_pad):
    def body(c_ref, f_ref, t1_ref, so_ref, si_ref):
        so = lax.rsqrt(jnp.maximum(c_ref[0], 1.0))
        si = lax.rsqrt(jnp.maximum(c_ref[1], 1.0))
        t1_ref[...] = f_ref[...] * so
        so_ref[...] = so
        si_ref[...] = si

    return pl.pallas_call(
        body,
        grid=(N_PAD // R_BLK,),
        in_specs=[
            pl.BlockSpec((NC, R_BLK, D), lambda i: (0, i, 0)),
            pl.BlockSpec((R_BLK, D), lambda i: (i, 0)),
        ],
        out_specs=[pl.BlockSpec((R_BLK, D), lambda i: (i, 0))] * 3,
        out_shape=[jax.ShapeDtypeStruct((N_PAD, D), jnp.float32)] * 3,
    )(cnt, feat_pad)


def _tc_dense(parts, sins, souts, W1, b1, W2):
    def body(p_ref, si_ref, so_ref, w1_ref, b1_ref, w2_ref, t2_ref):
        a = (p_ref[0] + p_ref[1]) * si_ref[...]
        h = jnp.dot(a, w1_ref[...], preferred_element_type=jnp.float32,
                    precision=lax.Precision.HIGHEST) + b1_ref[...]
        h = jnp.maximum(h, 0.0)
        so_wide = jnp.broadcast_to(so_ref[:, :1], h.shape)
        t2_ref[...] = jnp.dot(h * so_wide, w2_ref[...],
                              preferred_element_type=jnp.float32,
                              precision=lax.Precision.HIGHEST)

    dh = W1.shape[1]
    return pl.pallas_call(
        body,
        grid=(N_PAD // R_BLK,),
        in_specs=[
            pl.BlockSpec((NC, R_BLK, D), lambda i: (0, i, 0)),
            pl.BlockSpec((R_BLK, D), lambda i: (i, 0)),
            pl.BlockSpec((R_BLK, D), lambda i: (i, 0)),
            pl.BlockSpec((D, dh), lambda i: (0, 0)),
            pl.BlockSpec((1, dh), lambda i: (0, 0)),
            pl.BlockSpec((dh, D), lambda i: (0, 0)),
        ],
        out_specs=pl.BlockSpec((R_BLK, D), lambda i: (i, 0)),
        out_shape=jax.ShapeDtypeStruct((N_PAD, D), jnp.float32),
    )(parts, sins, souts, W1, b1.reshape(1, dh), W2)


def _tc_final(parts, sins, b2):
    def body(q_ref, si_ref, b2_ref, o_ref):
        q = (q_ref[0] + q_ref[1]) * si_ref[...]
        o_ref[...] = jnp.maximum(q + b2_ref[...], 0.0)

    return pl.pallas_call(
        body,
        grid=(N_PAD // R_BLK,),
        in_specs=[
            pl.BlockSpec((NC, R_BLK, D), lambda i: (0, i, 0)),
            pl.BlockSpec((R_BLK, D), lambda i: (i, 0)),
            pl.BlockSpec((1, D), lambda i: (0, 0)),
        ],
        out_specs=pl.BlockSpec((R_BLK, D), lambda i: (i, 0)),
        out_shape=jax.ShapeDtypeStruct((N_PAD, D), jnp.float32),
    )(parts, sins, b2.reshape(1, D))


def kernel(feat, edge_index, W1, b1, W2, b2):
    src = edge_index[0].astype(jnp.int32)
    dst = edge_index[1].astype(jnp.int32)
    e = src.shape[0]
    rows = -(-e // CHUNK)
    rpw = -(-rows // NW)
    rpw = -(-rpw // 4) * 4
    e_pad = rpw * NW * CHUNK
    pad = e_pad - e
    rows_pad = rpw * NW
    src_p = jnp.concatenate([src, jnp.full((pad,), N, jnp.int32)])
    src_p = src_p.reshape(rows_pad, CHUNK)
    dst_p = jnp.concatenate([dst, jnp.full((pad,), N, jnp.int32)])
    dst_p = dst_p.reshape(rows_pad, CHUNK)
    feat_p = jnp.pad(feat, ((0, N_PAD - N), (0, 0)))

    cnt = _make_deg_kernel(rpw * NW)(src_p, dst_p)
    t1, souts, sins = _tc_prep(cnt, feat_p)
    parts1 = _make_agg_kernel(rows_pad)(t1, src_p, dst_p)
    t2 = _tc_dense(parts1, sins, souts, W1, b1, W2)
    parts2 = _make_agg_kernel(rows_pad)(t2, src_p, dst_p)
    out = _tc_final(parts2, sins, b2)
    return out[:N]


# speedup vs baseline: 1.4823x; 1.4823x over previous
import functools

import jax
import jax.numpy as jnp
from jax import lax
from jax.experimental import pallas as pl
from jax.experimental.pallas import tpu as pltpu
from jax.experimental.pallas import tpu_sc as plsc

N = 10000
D = 128
NC = 2
NS = 16
NW = NC * NS
N_PAD = 10240
ROWS_PER_SUB = N_PAD // NS
CHUNK = 128
IBLK = 8
R_BLK = 512
FAST_CORE = 1
FAST_FRAC_NUM, FAST_FRAC_DEN = 17, 20


def _make_deg_kernel(total_rows):
    mesh = plsc.VectorSubcoreMesh(core_axis_name="c", subcore_axis_name="s")
    rows_per_sub = total_rows // NS

    @functools.partial(
        pl.kernel,
        mesh=mesh,
        out_type=jax.ShapeDtypeStruct((NC, N_PAD, D), jnp.float32),
        scratch_types=[
            pltpu.VMEM_SHARED((N_PAD, D), jnp.float32),
            pltpu.VMEM((CHUNK, D), jnp.float32),
            pltpu.VMEM((IBLK, CHUNK), jnp.int32),
            pltpu.SemaphoreType.DMA,
        ],
    )
    def deg_kernel(src_hbm, dst_hbm, cnt_hbm, cnt_sh, ones_v, idx_blk, ssem):
        c = lax.axis_index("c")
        s = lax.axis_index("s")

        def fill(val):
            def body(i, _):
                for k in range(D // 16):
                    ones_v[i, pl.ds(k * 16, 16)] = jnp.full((16,), val,
                                                            jnp.float32)
                return 0
            lax.fori_loop(0, CHUNK, body, 0)

        fill(0.0)
        base = s * ROWS_PER_SUB
        for k in range(ROWS_PER_SUB // CHUNK):
            pltpu.sync_copy(ones_v, cnt_sh.at[pl.ds(base + k * CHUNK, CHUNK)])
        fill(1.0)
        plsc.subcore_barrier()

        def count(edge_hbm):
            def body(b, _):
                rb = s * rows_per_sub + b * IBLK
                pltpu.sync_copy(edge_hbm.at[pl.ds(rb, IBLK)], idx_blk)
                cps = [pltpu.async_copy(ones_v, cnt_sh.at[idx_blk.at[j]],
                                        ssem, add=True)
                       for j in range(IBLK)]
                for cp in cps:
                    cp.wait()
                return 0
            lax.fori_loop(0, rows_per_sub // IBLK, body, 0)

        @pl.when(c == 0)
        def _():
            count(src_hbm)

        @pl.when(c == 1)
        def _():
            count(dst_hbm)

        plsc.subcore_barrier()
        for k in range(ROWS_PER_SUB // CHUNK):
            pltpu.sync_copy(cnt_sh.at[pl.ds(base + k * CHUNK, CHUNK)], ones_v)
            pltpu.sync_copy(ones_v,
                            cnt_hbm.at[c].at[pl.ds(base + k * CHUNK, CHUNK)])

    return deg_kernel


def _make_agg_kernel(total_rows):
    mesh = plsc.VectorSubcoreMesh(core_axis_name="c", subcore_axis_name="s")
    grain = NS * IBLK
    fast_total = (total_rows * FAST_FRAC_NUM // FAST_FRAC_DEN) // grain * grain
    f_t = fast_total // NS
    s_t = (total_rows - fast_total) // NS

    @functools.partial(
        pl.kernel,
        mesh=mesh,
        out_type=jax.ShapeDtypeStruct((NC, N_PAD, D), jnp.float32),
        scratch_types=[
            pltpu.VMEM_SHARED((N_PAD, D), jnp.float32),
            pltpu.VMEM((CHUNK, D), jnp.float32),
            pltpu.VMEM((CHUNK, D), jnp.float32),
            pltpu.VMEM((IBLK, CHUNK), jnp.int32),
            pltpu.VMEM((IBLK, CHUNK), jnp.int32),
            pltpu.SemaphoreType.DMA,
            pltpu.SemaphoreType.DMA,
            pltpu.SemaphoreType.DMA,
            pltpu.SemaphoreType.DMA,
        ],
    )
    def agg_kernel(table_hbm, src_hbm, dst_hbm, out_hbm,
                   acc_sh, rows0, rows1, sidx_blk, didx_blk,
                   gsem0, gsem1, ssem0, ssem1):
        c = lax.axis_index("c")
        s = lax.axis_index("s")
        my_rows = jnp.where(c == FAST_CORE, f_t, s_t)
        my_base = jnp.where(c == FAST_CORE, s * f_t, fast_total + s * s_t)

        def zrow(i, _):
            for k in range(D // 16):
                rows0[i, pl.ds(k * 16, 16)] = jnp.zeros((16,), jnp.float32)
            return 0
        lax.fori_loop(0, CHUNK, zrow, 0)

        base = s * ROWS_PER_SUB
        for k in range(ROWS_PER_SUB // CHUNK):
            pltpu.sync_copy(rows0, acc_sh.at[pl.ds(base + k * CHUNK, CHUNK)])
        plsc.subcore_barrier()

        def body(b, _):
            @pl.when(b > 0)
            def _():
                pltpu.make_async_copy(table_hbm.at[sidx_blk.at[0]], rows0,
                                      ssem0).wait()
                pltpu.make_async_copy(table_hbm.at[sidx_blk.at[0]], rows1,
                                      ssem1).wait()
            rb = my_base + b * IBLK
            pltpu.sync_copy(src_hbm.at[pl.ds(rb, IBLK)], sidx_blk)
            pltpu.sync_copy(dst_hbm.at[pl.ds(rb, IBLK)], didx_blk)
            for jj in range(IBLK // 2):
                j0, j1 = 2 * jj, 2 * jj + 1
                if jj > 0:
                    pltpu.make_async_copy(table_hbm.at[sidx_blk.at[j0]],
                                          rows0, ssem0).wait()
                    pltpu.make_async_copy(table_hbm.at[sidx_blk.at[j1]],
                                          rows1, ssem1).wait()
                cp0 = pltpu.async_copy(table_hbm.at[sidx_blk.at[j0]], rows0,
                                       gsem0)
                cp1 = pltpu.async_copy(table_hbm.at[sidx_blk.at[j1]], rows1,
                                       gsem1)
                cp0.wait()
                pltpu.async_copy(rows0, acc_sh.at[didx_blk.at[j0]], ssem0,
                                 add=True)
                cp1.wait()
                pltpu.async_copy(rows1, acc_sh.at[didx_blk.at[j1]], ssem1,
                                 add=True)
            return 0
        lax.fori_loop(0, my_rows // IBLK, body, 0)

        @pl.when(my_rows > 0)
        def _():
            pltpu.make_async_copy(table_hbm.at[sidx_blk.at[0]], rows0,
                                  ssem0).wait()
            pltpu.make_async_copy(table_hbm.at[sidx_blk.at[0]], rows1,
                                  ssem1).wait()
        plsc.subcore_barrier()

        for k in range(ROWS_PER_SUB // CHUNK):
            pltpu.sync_copy(acc_sh.at[pl.ds(base + k * CHUNK, CHUNK)], rows0)
            pltpu.sync_copy(rows0,
                            out_hbm.at[c].at[pl.ds(base + k * CHUNK, CHUNK)])

    return agg_kernel


def _tc_prep(cnt, feat_pad):
    def body(c_ref, f_ref, t1_ref, so_ref, si_ref):
        so = lax.rsqrt(jnp.maximum(c_ref[0], 1.0))
        si = lax.rsqrt(jnp.maximum(c_ref[1], 1.0))
        t1_ref[...] = f_ref[...] * so
        so_ref[...] = so
        si_ref[...] = si

    return pl.pallas_call(
        body,
        grid=(N_PAD // R_BLK,),
        in_specs=[
            pl.BlockSpec((NC, R_BLK, D), lambda i: (0, i, 0)),
            pl.BlockSpec((R_BLK, D), lambda i: (i, 0)),
        ],
        out_specs=[pl.BlockSpec((R_BLK, D), lambda i: (i, 0))] * 3,
        out_shape=[jax.ShapeDtypeStruct((N_PAD, D), jnp.float32)] * 3,
    )(cnt, feat_pad)


def _tc_dense(parts, sins, souts, W1, b1, W2):
    def body(p_ref, si_ref, so_ref, w1_ref, b1_ref, w2_ref, t2_ref):
        a = (p_ref[0] + p_ref[1]) * si_ref[...]
        h = jnp.dot(a, w1_ref[...], preferred_element_type=jnp.float32,
                    precision=lax.Precision.HIGHEST) + b1_ref[...]
        h = jnp.maximum(h, 0.0)
        so_wide = jnp.broadcast_to(so_ref[:, :1], h.shape)
        t2_ref[...] = jnp.dot(h * so_wide, w2_ref[...],
                              preferred_element_type=jnp.float32,
                              precision=lax.Precision.HIGHEST)

    dh = W1.shape[1]
    return pl.pallas_call(
        body,
        grid=(N_PAD // R_BLK,),
        in_specs=[
            pl.BlockSpec((NC, R_BLK, D), lambda i: (0, i, 0)),
            pl.BlockSpec((R_BLK, D), lambda i: (i, 0)),
            pl.BlockSpec((R_BLK, D), lambda i: (i, 0)),
            pl.BlockSpec((D, dh), lambda i: (0, 0)),
            pl.BlockSpec((1, dh), lambda i: (0, 0)),
            pl.BlockSpec((dh, D), lambda i: (0, 0)),
        ],
        out_specs=pl.BlockSpec((R_BLK, D), lambda i: (i, 0)),
        out_shape=jax.ShapeDtypeStruct((N_PAD, D), jnp.float32),
    )(parts, sins, souts, W1, b1.reshape(1, dh), W2)


def _tc_final(parts, sins, b2):
    def body(q_ref, si_ref, b2_ref, o_ref):
        q = (q_ref[0] + q_ref[1]) * si_ref[...]
        o_ref[...] = jnp.maximum(q + b2_ref[...], 0.0)

    return pl.pallas_call(
        body,
        grid=(N_PAD // R_BLK,),
        in_specs=[
            pl.BlockSpec((NC, R_BLK, D), lambda i: (0, i, 0)),
            pl.BlockSpec((R_BLK, D), lambda i: (i, 0)),
            pl.BlockSpec((1, D), lambda i: (0, 0)),
        ],
        out_specs=pl.BlockSpec((R_BLK, D), lambda i: (i, 0)),
        out_shape=jax.ShapeDtypeStruct((N_PAD, D), jnp.float32),
    )(parts, sins, b2.reshape(1, D))


def kernel(feat, edge_index, W1, b1, W2, b2):
    src = edge_index[0].astype(jnp.int32)
    dst = edge_index[1].astype(jnp.int32)
    e = src.shape[0]
    rows = -(-e // CHUNK)
    rpw = -(-rows // NW)
    rpw = -(-rpw // 4) * 4
    e_pad = rpw * NW * CHUNK
    pad = e_pad - e
    rows_pad = rpw * NW
    src_p = jnp.concatenate([src, jnp.full((pad,), N, jnp.int32)])
    src_p = src_p.reshape(rows_pad, CHUNK)
    dst_p = jnp.concatenate([dst, jnp.full((pad,), N, jnp.int32)])
    dst_p = dst_p.reshape(rows_pad, CHUNK)
    feat_p = jnp.pad(feat, ((0, N_PAD - N), (0, 0)))

    cnt = _make_deg_kernel(rpw * NW)(src_p, dst_p)
    t1, souts, sins = _tc_prep(cnt, feat_p)
    parts1 = _make_agg_kernel(rows_pad)(t1, src_p, dst_p)
    t2 = _tc_dense(parts1, sins, souts, W1, b1, W2)
    parts2 = _make_agg_kernel(rows_pad)(t2, src_p, dst_p)
    out = _tc_final(parts2, sins, b2)
    return out[:N]


# speedup vs baseline: 1.5411x; 1.0397x over previous
import functools

import jax
import jax.numpy as jnp
from jax import lax
from jax.experimental import pallas as pl
from jax.experimental.pallas import tpu as pltpu
from jax.experimental.pallas import tpu_sc as plsc

N = 10000
D = 128
NC = 2
NS = 16
NW = NC * NS
N_PAD = 10240
ROWS_PER_SUB = N_PAD // NS
CHUNK = 128
IBLK = 8
R_BLK = 512
FAST_CORE = 1
FAST_FRAC_NUM, FAST_FRAC_DEN = 19, 20


def _make_deg_kernel(total_rows):
    mesh = plsc.VectorSubcoreMesh(core_axis_name="c", subcore_axis_name="s")
    rows_per_sub = total_rows // NS

    @functools.partial(
        pl.kernel,
        mesh=mesh,
        out_type=jax.ShapeDtypeStruct((NC, N_PAD, D), jnp.float32),
        scratch_types=[
            pltpu.VMEM_SHARED((N_PAD, D), jnp.float32),
            pltpu.VMEM((CHUNK, D), jnp.float32),
            pltpu.VMEM((IBLK, CHUNK), jnp.int32),
            pltpu.SemaphoreType.DMA,
        ],
    )
    def deg_kernel(src_hbm, dst_hbm, cnt_hbm, cnt_sh, ones_v, idx_blk, ssem):
        c = lax.axis_index("c")
        s = lax.axis_index("s")

        def fill(val):
            def body(i, _):
                for k in range(D // 16):
                    ones_v[i, pl.ds(k * 16, 16)] = jnp.full((16,), val,
                                                            jnp.float32)
                return 0
            lax.fori_loop(0, CHUNK, body, 0)

        fill(0.0)
        base = s * ROWS_PER_SUB
        for k in range(ROWS_PER_SUB // CHUNK):
            pltpu.sync_copy(ones_v, cnt_sh.at[pl.ds(base + k * CHUNK, CHUNK)])
        fill(1.0)
        plsc.subcore_barrier()

        def count(edge_hbm):
            def body(b, _):
                rb = s * rows_per_sub + b * IBLK
                pltpu.sync_copy(edge_hbm.at[pl.ds(rb, IBLK)], idx_blk)
                cps = [pltpu.async_copy(ones_v, cnt_sh.at[idx_blk.at[j]],
                                        ssem, add=True)
                       for j in range(IBLK)]
                for cp in cps:
                    cp.wait()
                return 0
            lax.fori_loop(0, rows_per_sub // IBLK, body, 0)

        @pl.when(c == 0)
        def _():
            count(src_hbm)

        @pl.when(c == 1)
        def _():
            count(dst_hbm)

        plsc.subcore_barrier()
        for k in range(ROWS_PER_SUB // CHUNK):
            pltpu.sync_copy(cnt_sh.at[pl.ds(base + k * CHUNK, CHUNK)], ones_v)
            pltpu.sync_copy(ones_v,
                            cnt_hbm.at[c].at[pl.ds(base + k * CHUNK, CHUNK)])

    return deg_kernel


def _make_agg_kernel(total_rows):
    mesh = plsc.VectorSubcoreMesh(core_axis_name="c", subcore_axis_name="s")
    grain = NS * IBLK
    fast_total = (total_rows * FAST_FRAC_NUM // FAST_FRAC_DEN) // grain * grain
    f_t = fast_total // NS
    s_t = (total_rows - fast_total) // NS

    @functools.partial(
        pl.kernel,
        mesh=mesh,
        out_type=jax.ShapeDtypeStruct((NC, N_PAD, D), jnp.float32),
        scratch_types=[
            pltpu.VMEM_SHARED((N_PAD, D), jnp.float32),
            pltpu.VMEM((CHUNK, D), jnp.float32),
            pltpu.VMEM((CHUNK, D), jnp.float32),
            pltpu.VMEM((IBLK, CHUNK), jnp.int32),
            pltpu.VMEM((IBLK, CHUNK), jnp.int32),
            pltpu.SemaphoreType.DMA,
            pltpu.SemaphoreType.DMA,
            pltpu.SemaphoreType.DMA,
            pltpu.SemaphoreType.DMA,
        ],
    )
    def agg_kernel(table_hbm, src_hbm, dst_hbm, out_hbm,
                   acc_sh, rows0, rows1, sidx_blk, didx_blk,
                   gsem0, gsem1, ssem0, ssem1):
        c = lax.axis_index("c")
        s = lax.axis_index("s")
        my_rows = jnp.where(c == FAST_CORE, f_t, s_t)
        my_base = jnp.where(c == FAST_CORE, s * f_t, fast_total + s * s_t)

        def zrow(i, _):
            for k in range(D // 16):
                rows0[i, pl.ds(k * 16, 16)] = jnp.zeros((16,), jnp.float32)
            return 0
        lax.fori_loop(0, CHUNK, zrow, 0)

        base = s * ROWS_PER_SUB
        for k in range(ROWS_PER_SUB // CHUNK):
            pltpu.sync_copy(rows0, acc_sh.at[pl.ds(base + k * CHUNK, CHUNK)])
        plsc.subcore_barrier()

        def body(b, _):
            @pl.when(b > 0)
            def _():
                pltpu.make_async_copy(table_hbm.at[sidx_blk.at[0]], rows0,
                                      ssem0).wait()
                pltpu.make_async_copy(table_hbm.at[sidx_blk.at[0]], rows1,
                                      ssem1).wait()
            rb = my_base + b * IBLK
            pltpu.sync_copy(src_hbm.at[pl.ds(rb, IBLK)], sidx_blk)
            pltpu.sync_copy(dst_hbm.at[pl.ds(rb, IBLK)], didx_blk)
            for jj in range(IBLK // 2):
                j0, j1 = 2 * jj, 2 * jj + 1
                if jj > 0:
                    pltpu.make_async_copy(table_hbm.at[sidx_blk.at[j0]],
                                          rows0, ssem0).wait()
                    pltpu.make_async_copy(table_hbm.at[sidx_blk.at[j1]],
                                          rows1, ssem1).wait()
                cp0 = pltpu.async_copy(table_hbm.at[sidx_blk.at[j0]], rows0,
                                       gsem0)
                cp1 = pltpu.async_copy(table_hbm.at[sidx_blk.at[j1]], rows1,
                                       gsem1)
                cp0.wait()
                pltpu.async_copy(rows0, acc_sh.at[didx_blk.at[j0]], ssem0,
                                 add=True)
                cp1.wait()
                pltpu.async_copy(rows1, acc_sh.at[didx_blk.at[j1]], ssem1,
                                 add=True)
            return 0
        lax.fori_loop(0, my_rows // IBLK, body, 0)

        @pl.when(my_rows > 0)
        def _():
            pltpu.make_async_copy(table_hbm.at[sidx_blk.at[0]], rows0,
                                  ssem0).wait()
            pltpu.make_async_copy(table_hbm.at[sidx_blk.at[0]], rows1,
                                  ssem1).wait()
        plsc.subcore_barrier()

        for k in range(ROWS_PER_SUB // CHUNK):
            pltpu.sync_copy(acc_sh.at[pl.ds(base + k * CHUNK, CHUNK)], rows0)
            pltpu.sync_copy(rows0,
                            out_hbm.at[c].at[pl.ds(base + k * CHUNK, CHUNK)])

    return agg_kernel


def _tc_prep(cnt, feat_pad):
    def body(c_ref, f_ref, t1_ref, so_ref, si_ref):
        so = lax.rsqrt(jnp.maximum(c_ref[0], 1.0))
        si = lax.rsqrt(jnp.maximum(c_ref[1], 1.0))
        t1_ref[...] = f_ref[...] * so
        so_ref[...] = so
        si_ref[...] = si

    return pl.pallas_call(
        body,
        grid=(N_PAD // R_BLK,),
        in_specs=[
            pl.BlockSpec((NC, R_BLK, D), lambda i: (0, i, 0)),
            pl.BlockSpec((R_BLK, D), lambda i: (i, 0)),
        ],
        out_specs=[pl.BlockSpec((R_BLK, D), lambda i: (i, 0))] * 3,
        out_shape=[jax.ShapeDtypeStruct((N_PAD, D), jnp.float32)] * 3,
    )(cnt, feat_pad)


def _tc_dense(parts, sins, souts, W1, b1, W2):
    def body(p_ref, si_ref, so_ref, w1_ref, b1_ref, w2_ref, t2_ref):
        a = (p_ref[0] + p_ref[1]) * si_ref[...]
        h = jnp.dot(a, w1_ref[...], preferred_element_type=jnp.float32,
                    precision=lax.Precision.HIGHEST) + b1_ref[...]
        h = jnp.maximum(h, 0.0)
        so_wide = jnp.broadcast_to(so_ref[:, :1], h.shape)
        t2_ref[...] = jnp.dot(h * so_wide, w2_ref[...],
                              preferred_element_type=jnp.float32,
                              precision=lax.Precision.HIGHEST)

    dh = W1.shape[1]
    return pl.pallas_call(
        body,
        grid=(N_PAD // R_BLK,),
        in_specs=[
            pl.BlockSpec((NC, R_BLK, D), lambda i: (0, i, 0)),
            pl.BlockSpec((R_BLK, D), lambda i: (i, 0)),
            pl.BlockSpec((R_BLK, D), lambda i: (i, 0)),
            pl.BlockSpec((D, dh), lambda i: (0, 0)),
            pl.BlockSpec((1, dh), lambda i: (0, 0)),
            pl.BlockSpec((dh, D), lambda i: (0, 0)),
        ],
        out_specs=pl.BlockSpec((R_BLK, D), lambda i: (i, 0)),
        out_shape=jax.ShapeDtypeStruct((N_PAD, D), jnp.float32),
    )(parts, sins, souts, W1, b1.reshape(1, dh), W2)


def _tc_final(parts, sins, b2):
    def body(q_ref, si_ref, b2_ref, o_ref):
        q = (q_ref[0] + q_ref[1]) * si_ref[...]
        o_ref[...] = jnp.maximum(q + b2_ref[...], 0.0)

    return pl.pallas_call(
        body,
        grid=(N_PAD // R_BLK,),
        in_specs=[
            pl.BlockSpec((NC, R_BLK, D), lambda i: (0, i, 0)),
            pl.BlockSpec((R_BLK, D), lambda i: (i, 0)),
            pl.BlockSpec((1, D), lambda i: (0, 0)),
        ],
        out_specs=pl.BlockSpec((R_BLK, D), lambda i: (i, 0)),
        out_shape=jax.ShapeDtypeStruct((N_PAD, D), jnp.float32),
    )(parts, sins, b2.reshape(1, D))


def kernel(feat, edge_index, W1, b1, W2, b2):
    src = edge_index[0].astype(jnp.int32)
    dst = edge_index[1].astype(jnp.int32)
    e = src.shape[0]
    rows = -(-e // CHUNK)
    rpw = -(-rows // NW)
    rpw = -(-rpw // 4) * 4
    e_pad = rpw * NW * CHUNK
    pad = e_pad - e
    rows_pad = rpw * NW
    src_p = jnp.concatenate([src, jnp.full((pad,), N, jnp.int32)])
    src_p = src_p.reshape(rows_pad, CHUNK)
    dst_p = jnp.concatenate([dst, jnp.full((pad,), N, jnp.int32)])
    dst_p = dst_p.reshape(rows_pad, CHUNK)
    feat_p = jnp.pad(feat, ((0, N_PAD - N), (0, 0)))

    cnt = _make_deg_kernel(rpw * NW)(src_p, dst_p)
    t1, souts, sins = _tc_prep(cnt, feat_p)
    parts1 = _make_agg_kernel(rows_pad)(t1, src_p, dst_p)
    t2 = _tc_dense(parts1, sins, souts, W1, b1, W2)
    parts2 = _make_agg_kernel(rows_pad)(t2, src_p, dst_p)
    out = _tc_final(parts2, sins, b2)
    return out[:N]
